# baseline (speedup 1.0000x reference)
import functools

import jax
import jax.numpy as jnp
from jax import lax
from jax.experimental import pallas as pl
from jax.experimental.pallas import tpu as pltpu
from jax.experimental.pallas import tpu_sc as plsc

S, GN = 512, 256
N_COMMANDS = 7
ARGS_DIM = 256
N_ARGS = 11
D_MODEL = 256
ARG_EMB_ROWS = ARGS_DIM + 1
GROUP_ROWS = 10

T = S * GN
SEG = 264
CG_OFF = SEG * N_ARGS
RTAB = CG_OFF + 72
NCOL = N_ARGS + 1

NC, NS = 2, 16
NW = NC * NS
TW = T // NW
C = 16
NCHUNK = TW // C
SUP = 1024
CPS = SUP // C


def _tab_body(ce_ref, aep_ref, w_ref, b_ref, ge_ref, out_ref):
    f32 = jnp.float32
    for j in range(N_ARGS):
        w_j = w_ref[:, j * 64:(j + 1) * 64]
        t_j = jax.lax.dot_general(
            aep_ref[...], w_j, (((1,), (1,)), ((), ())),
            preferred_element_type=f32)
        out_ref[j * SEG:(j + 1) * SEG, :] = t_j
    ioc = jax.lax.broadcasted_iota(jnp.int32, (72, N_COMMANDS), 1)
    row = jax.lax.broadcasted_iota(jnp.int32, (72, N_COMMANDS), 0)
    cg = jnp.dot((row // GROUP_ROWS == ioc).astype(f32), ce_ref[...],
                 preferred_element_type=f32)
    iog = jax.lax.broadcasted_iota(jnp.int32, (72, GROUP_ROWS), 1)
    rowg = jax.lax.broadcasted_iota(jnp.int32, (72, GROUP_ROWS), 0)
    cg = cg + jnp.dot((rowg % GROUP_ROWS == iog).astype(f32), ge_ref[...],
                      preferred_element_type=f32)
    out_ref[CG_OFF:CG_OFF + 72, :] = cg + b_ref[...]


def _build_table(command_embed, arg_embed, W_fcn, b_fcn, group_embed):
    aep = jnp.zeros((SEG, 64), jnp.float32).at[:ARG_EMB_ROWS].set(arg_embed)
    return pl.pallas_call(
        _tab_body,
        out_shape=jax.ShapeDtypeStruct((RTAB, D_MODEL), jnp.float32),
    )(command_embed, aep, W_fcn, b_fcn.reshape(1, D_MODEL), group_embed)


def _sc_body(tab, raw, out, rawv, idxb, gb, obuf, semg0, semg1, semr):
    cid = lax.axis_index("c")
    sid = lax.axis_index("s")
    wid = sid * NC + cid
    tok0 = wid * TW
    semg = (semg0, semg1)

    def stage_super(sidx):
        pltpu.async_copy(raw.at[:, pl.ds(tok0 + sidx * SUP, SUP)], rawv,
                         semr).wait()

    def build_idx(pb, n):
        off = lax.rem(n, CPS) * C
        for q in range(C // 16):
            sl = pl.ds(off + q * 16, 16)
            dsl = pl.ds(q * 16, 16)
            for j in range(N_ARGS):
                idxb[pb, j, dsl] = rawv[j, sl] + (j * SEG + 1)
            idxb[pb, N_ARGS, dsl] = (rawv[N_ARGS, sl] * GROUP_ROWS
                                     + rawv[N_ARGS + 1, sl] + CG_OFF)

    def fire(pb):
        for col in range(NCOL):
            pltpu.async_copy(tab.at[idxb.at[pb, col]], gb.at[pb, col],
                             semg[pb])

    def drain(pb):
        for col in range(NCOL):
            pltpu.make_async_copy(tab.at[pl.ds(0, C)], gb.at[pb, col],
                                  semg[pb]).wait()

    def accum(pb, n):
        def tok(t, _):
            for d in range(D_MODEL // 16):
                sl = pl.ds(d * 16, 16)
                v = gb[pb, 0, t, sl]
                for col in range(1, NCOL):
                    v = v + gb[pb, col, t, sl]
                obuf[t, sl] = v
            return ()
        lax.fori_loop(0, C, tok, ())
        pltpu.sync_copy(obuf, out.at[pl.ds(tok0 + n * C, C)])

    stage_super(0)
    build_idx(0, 0)
    fire(0)

    def body(k, _):
        i = 2 * k
        build_idx(1, i + 1)
        fire(1)
        drain(0)
        accum(0, i)

        @pl.when(i + 2 < NCHUNK)
        def _():
            @pl.when(lax.rem(i + 2, CPS) == 0)
            def _():
                stage_super((i + 2) // CPS)
            build_idx(0, i + 2)
            fire(0)

        drain(1)
        accum(1, i + 1)
        return ()

    lax.fori_loop(0, NCHUNK // 2, body, ())


SB = 8
NT = SB * GN


def _epi_body(acc_ref, pe_ref, out_ref):
    f32 = jnp.float32
    tok_row = jax.lax.broadcasted_iota(jnp.int32, (NT, SB), 0) // GN
    rep = (tok_row == jax.lax.broadcasted_iota(jnp.int32, (NT, SB), 1)).astype(f32)
    out_ref[...] = acc_ref[...].astype(f32) + jnp.dot(
        rep, pe_ref[...], preferred_element_type=f32)


def _epilogue(accsum, pos_embed):
    return pl.pallas_call(
        _epi_body,
        grid=(S // SB,),
        in_specs=[
            pl.BlockSpec((NT, D_MODEL), lambda i: (i, 0)),
            pl.BlockSpec((SB, D_MODEL), lambda i: (i, 0)),
        ],
        out_specs=pl.BlockSpec((NT, D_MODEL), lambda i: (i, 0)),
        out_shape=jax.ShapeDtypeStruct((T, D_MODEL), jnp.float32),
    )(accsum, pos_embed)


def kernel(commands, args, groups, command_embed, arg_embed, W_fcn, b_fcn, group_embed, pos_embed):
    cmdf = commands.astype(jnp.int32).reshape(1, T)
    grpf = groups.astype(jnp.int32).reshape(1, T)
    argsT = args.astype(jnp.int32).reshape(T, N_ARGS).T.reshape(N_ARGS, T)
    raw = jnp.concatenate([argsT, cmdf, grpf], axis=0)

    tab = _build_table(command_embed, arg_embed, W_fcn, b_fcn, group_embed)

    mesh = plsc.VectorSubcoreMesh(core_axis_name="c", subcore_axis_name="s",
                                  num_cores=NC)
    sc = pl.kernel(
        _sc_body,
        out_type=jax.ShapeDtypeStruct((T, D_MODEL), jnp.float32),
        mesh=mesh,
        scratch_types=[
            pltpu.VMEM((N_ARGS + 2, SUP), jnp.int32),
            pltpu.VMEM((2, NCOL, C), jnp.int32),
            pltpu.VMEM((2, NCOL, C, D_MODEL), jnp.float32),
            pltpu.VMEM((C, D_MODEL), jnp.float32),
            pltpu.SemaphoreType.DMA,
            pltpu.SemaphoreType.DMA,
            pltpu.SemaphoreType.DMA,
        ],
    )
    accsum = sc(tab, raw)
    out = _epilogue(accsum, pos_embed)
    return out.reshape(S, GN, D_MODEL)


# speedup vs baseline: 1.0509x; 1.0509x over previous
import functools

import jax
import jax.numpy as jnp
from jax import lax
from jax.experimental import pallas as pl
from jax.experimental.pallas import tpu as pltpu
from jax.experimental.pallas import tpu_sc as plsc

S, GN = 512, 256
N_COMMANDS = 7
ARGS_DIM = 256
N_ARGS = 11
D_MODEL = 256
ARG_EMB_ROWS = ARGS_DIM + 1
GROUP_ROWS = 10

T = S * GN
SEG = 264
CG_OFF = SEG * N_ARGS
RTAB = CG_OFF + 72
NCOL = N_ARGS + 1

NC, NS = 2, 16
NW = NC * NS
TW = T // NW
C = 16
NCHUNK = TW // C
SUP = 1024
CPS = SUP // C


def _tab_body(ce_ref, aep_ref, w_ref, b_ref, ge_ref, out_ref):
    f32 = jnp.float32
    for j in range(N_ARGS):
        w_j = w_ref[:, j * 64:(j + 1) * 64]
        t_j = jax.lax.dot_general(
            aep_ref[...], w_j, (((1,), (1,)), ((), ())),
            preferred_element_type=f32)
        out_ref[j * SEG:(j + 1) * SEG, :] = t_j
    ioc = jax.lax.broadcasted_iota(jnp.int32, (72, N_COMMANDS), 1)
    row = jax.lax.broadcasted_iota(jnp.int32, (72, N_COMMANDS), 0)
    cg = jnp.dot((row // GROUP_ROWS == ioc).astype(f32), ce_ref[...],
                 preferred_element_type=f32)
    iog = jax.lax.broadcasted_iota(jnp.int32, (72, GROUP_ROWS), 1)
    rowg = jax.lax.broadcasted_iota(jnp.int32, (72, GROUP_ROWS), 0)
    cg = cg + jnp.dot((rowg % GROUP_ROWS == iog).astype(f32), ge_ref[...],
                      preferred_element_type=f32)
    out_ref[CG_OFF:CG_OFF + 72, :] = cg + b_ref[...]


def _build_table(command_embed, arg_embed, W_fcn, b_fcn, group_embed):
    aep = jnp.zeros((SEG, 64), jnp.float32).at[:ARG_EMB_ROWS].set(arg_embed)
    return pl.pallas_call(
        _tab_body,
        out_shape=jax.ShapeDtypeStruct((RTAB, D_MODEL), jnp.float32),
    )(command_embed, aep, W_fcn, b_fcn.reshape(1, D_MODEL), group_embed)


def _sc_body(tab, raw, out, rawv, idxb, gb, obuf, semg0, semg1, semr, semo0, semo1):
    cid = lax.axis_index("c")
    sid = lax.axis_index("s")
    wid = sid * NC + cid
    tok0 = wid * TW
    semg = (semg0, semg1)

    def stage_super(sidx):
        pltpu.async_copy(raw.at[:, pl.ds(tok0 + sidx * SUP, SUP)], rawv,
                         semr).wait()

    def build_idx(pb, n):
        off = lax.rem(n, CPS) * C
        for q in range(C // 16):
            sl = pl.ds(off + q * 16, 16)
            dsl = pl.ds(q * 16, 16)
            for j in range(N_ARGS):
                idxb[pb, j, dsl] = rawv[j, sl] + (j * SEG + 1)
            idxb[pb, N_ARGS, dsl] = (rawv[N_ARGS, sl] * GROUP_ROWS
                                     + rawv[N_ARGS + 1, sl] + CG_OFF)

    def fire(pb):
        for col in range(NCOL):
            pltpu.async_copy(tab.at[idxb.at[pb, col]], gb.at[pb, col],
                             semg[pb])

    def drain(pb):
        for col in range(NCOL):
            pltpu.make_async_copy(tab.at[pl.ds(0, C)], gb.at[pb, col],
                                  semg[pb]).wait()

    semo = (semo0, semo1)

    def accum(pb, n):
        def tok(t, _):
            for d in range(D_MODEL // 16):
                sl = pl.ds(d * 16, 16)
                v = gb[pb, 0, t, sl]
                for col in range(1, NCOL):
                    v = v + gb[pb, col, t, sl]
                obuf[pb, t, sl] = v
            return ()
        lax.fori_loop(0, C, tok, ())
        pltpu.async_copy(obuf.at[pb], out.at[pl.ds(tok0 + n * C, C)],
                         semo[pb])

    def drain_out(pb):
        pltpu.make_async_copy(out.at[pl.ds(tok0, C)], obuf.at[pb],
                              semo[pb]).wait()

    stage_super(0)
    build_idx(0, 0)
    fire(0)

    def body(k, _):
        i = 2 * k
        build_idx(1, i + 1)
        fire(1)
        drain(0)

        @pl.when(i > 0)
        def _():
            drain_out(0)
        accum(0, i)

        @pl.when(i + 2 < NCHUNK)
        def _():
            @pl.when(lax.rem(i + 2, CPS) == 0)
            def _():
                stage_super((i + 2) // CPS)
            build_idx(0, i + 2)
            fire(0)

        drain(1)

        @pl.when(i > 0)
        def _():
            drain_out(1)
        accum(1, i + 1)
        return ()

    lax.fori_loop(0, NCHUNK // 2, body, ())
    drain_out(0)
    drain_out(1)


SB = 8
NT = SB * GN


def _epi_body(acc_ref, pe_ref, out_ref):
    f32 = jnp.float32
    tok_row = jax.lax.broadcasted_iota(jnp.int32, (NT, SB), 0) // GN
    rep = (tok_row == jax.lax.broadcasted_iota(jnp.int32, (NT, SB), 1)).astype(f32)
    out_ref[...] = acc_ref[...].astype(f32) + jnp.dot(
        rep, pe_ref[...], preferred_element_type=f32)


def _epilogue(accsum, pos_embed):
    return pl.pallas_call(
        _epi_body,
        grid=(S // SB,),
        in_specs=[
            pl.BlockSpec((NT, D_MODEL), lambda i: (i, 0)),
            pl.BlockSpec((SB, D_MODEL), lambda i: (i, 0)),
        ],
        out_specs=pl.BlockSpec((NT, D_MODEL), lambda i: (i, 0)),
        out_shape=jax.ShapeDtypeStruct((T, D_MODEL), jnp.float32),
    )(accsum, pos_embed)


def kernel(commands, args, groups, command_embed, arg_embed, W_fcn, b_fcn, group_embed, pos_embed):
    cmdf = commands.astype(jnp.int32).reshape(1, T)
    grpf = groups.astype(jnp.int32).reshape(1, T)
    argsT = args.astype(jnp.int32).reshape(T, N_ARGS).T.reshape(N_ARGS, T)
    raw = jnp.concatenate([argsT, cmdf, grpf], axis=0)

    tab = _build_table(command_embed, arg_embed, W_fcn, b_fcn, group_embed)

    mesh = plsc.VectorSubcoreMesh(core_axis_name="c", subcore_axis_name="s",
                                  num_cores=NC)
    sc = pl.kernel(
        _sc_body,
        out_type=jax.ShapeDtypeStruct((T, D_MODEL), jnp.float32),
        mesh=mesh,
        scratch_types=[
            pltpu.VMEM((N_ARGS + 2, SUP), jnp.int32),
            pltpu.VMEM((2, NCOL, C), jnp.int32),
            pltpu.VMEM((2, NCOL, C, D_MODEL), jnp.float32),
            pltpu.VMEM((2, C, D_MODEL), jnp.float32),
            pltpu.SemaphoreType.DMA,
            pltpu.SemaphoreType.DMA,
            pltpu.SemaphoreType.DMA,
            pltpu.SemaphoreType.DMA,
            pltpu.SemaphoreType.DMA,
        ],
    )
    accsum = sc(tab, raw)
    out = _epilogue(accsum, pos_embed)
    return out.reshape(S, GN, D_MODEL)


# speedup vs baseline: 1.8831x; 1.7918x over previous
import functools

import jax
import jax.numpy as jnp
from jax import lax
from jax.experimental import pallas as pl
from jax.experimental.pallas import tpu as pltpu
from jax.experimental.pallas import tpu_sc as plsc

S, GN = 512, 256
N_COMMANDS = 7
ARGS_DIM = 256
N_ARGS = 11
D_MODEL = 256
ARG_EMB_ROWS = ARGS_DIM + 1
GROUP_ROWS = 10

T = S * GN
S_TC = 320
S_SC = S - S_TC
T_TC = S_TC * GN
T_SC = S_SC * GN
SEG = 264
CG_OFF = SEG * N_ARGS
RTAB = CG_OFF + 72
NCOL = N_ARGS + 1

NC, NS = 2, 16
NW = NC * NS
TW = T_SC // NW
C = 16
NCHUNK = TW // C
SUP = 512
CPS = SUP // C


def _tab_body(ce_ref, aep_ref, w_ref, b_ref, ge_ref, out_ref):
    f32 = jnp.float32
    for j in range(N_ARGS):
        w_j = w_ref[:, j * 64:(j + 1) * 64]
        t_j = jax.lax.dot_general(
            aep_ref[...], w_j, (((1,), (1,)), ((), ())),
            preferred_element_type=f32)
        out_ref[j * SEG:(j + 1) * SEG, :] = t_j
    ioc = jax.lax.broadcasted_iota(jnp.int32, (72, N_COMMANDS), 1)
    row = jax.lax.broadcasted_iota(jnp.int32, (72, N_COMMANDS), 0)
    cg = jnp.dot((row // GROUP_ROWS == ioc).astype(f32), ce_ref[...],
                 preferred_element_type=f32)
    iog = jax.lax.broadcasted_iota(jnp.int32, (72, GROUP_ROWS), 1)
    rowg = jax.lax.broadcasted_iota(jnp.int32, (72, GROUP_ROWS), 0)
    cg = cg + jnp.dot((rowg % GROUP_ROWS == iog).astype(f32), ge_ref[...],
                      preferred_element_type=f32)
    out_ref[CG_OFF:CG_OFF + 72, :] = cg + b_ref[...]


def _build_table(command_embed, arg_embed, W_fcn, b_fcn, group_embed):
    aep = jnp.zeros((SEG, 64), jnp.float32).at[:ARG_EMB_ROWS].set(arg_embed)
    return pl.pallas_call(
        _tab_body,
        out_shape=jax.ShapeDtypeStruct((RTAB, D_MODEL), jnp.float32),
    )(command_embed, aep, W_fcn, b_fcn.reshape(1, D_MODEL), group_embed)


def _sc_body(tab, raw, out, rawv, idxb, gb, obuf, semg0, semg1, semr, semo0, semo1):
    cid = lax.axis_index("c")
    sid = lax.axis_index("s")
    wid = sid * NC + cid
    tok0 = wid * TW
    semg = (semg0, semg1)

    def stage_super(sidx):
        pltpu.async_copy(raw.at[:, pl.ds(tok0 + sidx * SUP, SUP)], rawv,
                         semr).wait()

    def build_idx(pb, n):
        off = lax.rem(n, CPS) * C
        for q in range(C // 16):
            sl = pl.ds(off + q * 16, 16)
            dsl = pl.ds(q * 16, 16)
            for j in range(N_ARGS):
                idxb[pb, j, dsl] = rawv[j, sl] + (j * SEG + 1)
            idxb[pb, N_ARGS, dsl] = (rawv[N_ARGS, sl] * GROUP_ROWS
                                     + rawv[N_ARGS + 1, sl] + CG_OFF)

    def fire(pb):
        for col in range(NCOL):
            pltpu.async_copy(tab.at[idxb.at[pb, col]], gb.at[pb, col],
                             semg[pb])

    def drain(pb):
        for col in range(NCOL):
            pltpu.make_async_copy(tab.at[pl.ds(0, C)], gb.at[pb, col],
                                  semg[pb]).wait()

    semo = (semo0, semo1)

    def accum(pb, n):
        def tok(t, _):
            for d in range(D_MODEL // 16):
                sl = pl.ds(d * 16, 16)
                v = gb[pb, 0, t, sl]
                for col in range(1, NCOL):
                    v = v + gb[pb, col, t, sl]
                obuf[pb, t, sl] = v
            return ()
        lax.fori_loop(0, C, tok, ())
        pltpu.async_copy(obuf.at[pb], out.at[pl.ds(tok0 + n * C, C)],
                         semo[pb])

    def drain_out(pb):
        pltpu.make_async_copy(out.at[pl.ds(tok0, C)], obuf.at[pb],
                              semo[pb]).wait()

    stage_super(0)
    build_idx(0, 0)
    fire(0)

    def body(k, _):
        i = 2 * k
        build_idx(1, i + 1)
        fire(1)
        drain(0)

        @pl.when(i > 0)
        def _():
            drain_out(0)
        accum(0, i)

        @pl.when(i + 2 < NCHUNK)
        def _():
            @pl.when(lax.rem(i + 2, CPS) == 0)
            def _():
                stage_super((i + 2) // CPS)
            build_idx(0, i + 2)
            fire(0)

        drain(1)

        @pl.when(i > 0)
        def _():
            drain_out(1)
        accum(1, i + 1)
        return ()

    lax.fori_loop(0, NCHUNK // 2, body, ())
    drain_out(0)
    drain_out(1)


SB = 8
NT = SB * GN


def _epi_body(acc_ref, pe_ref, out_ref):
    f32 = jnp.float32
    tok_row = jax.lax.broadcasted_iota(jnp.int32, (NT, SB), 0) // GN
    rep = (tok_row == jax.lax.broadcasted_iota(jnp.int32, (NT, SB), 1)).astype(f32)
    out_ref[...] = acc_ref[...].astype(f32) + jnp.dot(
        rep, pe_ref[...], preferred_element_type=f32)


def _epilogue(accsum, pos_embed):
    return pl.pallas_call(
        _epi_body,
        grid=(S_SC // SB,),
        in_specs=[
            pl.BlockSpec((NT, D_MODEL), lambda i: (i, 0)),
            pl.BlockSpec((SB, D_MODEL), lambda i: (i, 0)),
        ],
        out_specs=pl.BlockSpec((NT, D_MODEL), lambda i: (i, 0)),
        out_shape=jax.ShapeDtypeStruct((T_SC, D_MODEL), jnp.float32),
    )(accsum, pos_embed)


def _oh_body(cmd_ref, args_ref, grp_ref, ce_ref, ae_ref, w_ref, b_ref, ge_ref, pe_ref, out_ref):
    f32 = jnp.float32
    bf16 = jnp.bfloat16
    cmd = cmd_ref[...]
    grp = grp_ref[...]
    args = args_ref[...]

    ioc = jax.lax.broadcasted_iota(jnp.int32, (NT, N_COMMANDS), 1)
    acc = jnp.dot((ioc == cmd).astype(bf16), ce_ref[...],
                  preferred_element_type=f32)
    iog = jax.lax.broadcasted_iota(jnp.int32, (NT, GROUP_ROWS), 1)
    acc = acc + jnp.dot((iog == grp).astype(bf16), ge_ref[...],
                        preferred_element_type=f32)
    tok_row = jax.lax.broadcasted_iota(jnp.int32, (NT, SB), 0) // GN
    rep = (tok_row == jax.lax.broadcasted_iota(jnp.int32, (NT, SB), 1)).astype(bf16)
    acc = acc + jnp.dot(rep, pe_ref[...], preferred_element_type=f32)

    ioa = jax.lax.broadcasted_iota(jnp.int32, (NT, ARG_EMB_ROWS), 1)
    a_parts = []
    for j in range(N_ARGS):
        aj = args[:, j:j + 1] + 1
        oh = (ioa == aj).astype(bf16)
        a_parts.append(jnp.dot(oh, ae_ref[...], preferred_element_type=f32))
    a = jnp.concatenate(a_parts, axis=1).astype(bf16)
    acc = acc + jax.lax.dot_general(
        a, w_ref[...], (((1,), (1,)), ((), ())),
        preferred_element_type=f32)
    out_ref[...] = acc + b_ref[...]


def _tc_tokens(cmdf, argsf, grpf, ce, ae, w, bf, ge, pe):
    return pl.pallas_call(
        _oh_body,
        grid=(S_TC // SB,),
        in_specs=[
            pl.BlockSpec((NT, 1), lambda i: (i, 0)),
            pl.BlockSpec((NT, N_ARGS), lambda i: (i, 0)),
            pl.BlockSpec((NT, 1), lambda i: (i, 0)),
            pl.BlockSpec((N_COMMANDS, D_MODEL), lambda i: (0, 0)),
            pl.BlockSpec((ARG_EMB_ROWS, 64), lambda i: (0, 0)),
            pl.BlockSpec((D_MODEL, 64 * N_ARGS), lambda i: (0, 0)),
            pl.BlockSpec((1, D_MODEL), lambda i: (0, 0)),
            pl.BlockSpec((GROUP_ROWS, D_MODEL), lambda i: (0, 0)),
            pl.BlockSpec((SB, D_MODEL), lambda i: (i, 0)),
        ],
        out_specs=pl.BlockSpec((NT, D_MODEL), lambda i: (i, 0)),
        out_shape=jax.ShapeDtypeStruct((T_TC, D_MODEL), jnp.float32),
    )(cmdf, argsf, grpf, ce, ae, w, bf, ge, pe)


def kernel(commands, args, groups, command_embed, arg_embed, W_fcn, b_fcn, group_embed, pos_embed):
    commands = commands.astype(jnp.int32)
    args = args.astype(jnp.int32)
    groups = groups.astype(jnp.int32)

    cmd_sc = commands[S_TC:].reshape(1, T_SC)
    grp_sc = groups[S_TC:].reshape(1, T_SC)
    args_sc = args[S_TC:].reshape(T_SC, N_ARGS).T.reshape(N_ARGS, T_SC)
    raw = jnp.concatenate([args_sc, cmd_sc, grp_sc], axis=0)

    tab = _build_table(command_embed, arg_embed, W_fcn, b_fcn, group_embed)

    mesh = plsc.VectorSubcoreMesh(core_axis_name="c", subcore_axis_name="s",
                                  num_cores=NC)
    sc = pl.kernel(
        _sc_body,
        out_type=jax.ShapeDtypeStruct((T_SC, D_MODEL), jnp.float32),
        mesh=mesh,
        scratch_types=[
            pltpu.VMEM((N_ARGS + 2, SUP), jnp.int32),
            pltpu.VMEM((2, NCOL, C), jnp.int32),
            pltpu.VMEM((2, NCOL, C, D_MODEL), jnp.float32),
            pltpu.VMEM((2, C, D_MODEL), jnp.float32),
            pltpu.SemaphoreType.DMA,
            pltpu.SemaphoreType.DMA,
            pltpu.SemaphoreType.DMA,
            pltpu.SemaphoreType.DMA,
            pltpu.SemaphoreType.DMA,
        ],
    )
    accsum = sc(tab, raw)
    out_sc = _epilogue(accsum, pos_embed[S_TC:])

    out_tc = _tc_tokens(
        commands[:S_TC].reshape(T_TC, 1),
        args[:S_TC].reshape(T_TC, N_ARGS),
        groups[:S_TC].reshape(T_TC, 1),
        command_embed.astype(jnp.bfloat16),
        arg_embed.astype(jnp.bfloat16),
        W_fcn.astype(jnp.bfloat16),
        b_fcn.reshape(1, D_MODEL),
        group_embed.astype(jnp.bfloat16),
        pos_embed[:S_TC].astype(jnp.bfloat16),
    )

    out = jnp.concatenate([out_tc, out_sc], axis=0)
    return out.reshape(S, GN, D_MODEL)
